# Initial kernel scaffold; baseline (speedup 1.0000x reference)
import jax
import jax.numpy as jnp
from jax.experimental import pallas as pl


def kernel(features, edge_index, W1, b1, gamma1, beta1, W2, b2, gamma2, beta2, Wr, br):
    raise NotImplementedError("write your pallas kernel here")


# speedup vs baseline: 1.4808x; 1.4808x over previous
import functools

import jax
import jax.numpy as jnp
from jax import lax
from jax.experimental import pallas as pl
from jax.experimental.pallas import tpu as pltpu
from jax.experimental.pallas import tpu_sc as plsc

N = 10000
NP = 10240
E = 160000
NTILES = 32
NCHUNK = 48
CB = 128
EP = NTILES * NCHUNK * CB
DEGW = 16
ROWS_PER_SUB = NP // 16

_mesh = plsc.VectorSubcoreMesh(core_axis_name="c", subcore_axis_name="s")


def _wid(c, s):
    return s * 2 + c


@functools.partial(
    pl.kernel,
    mesh=_mesh,
    out_type=jax.ShapeDtypeStruct((2, 2, NP, DEGW), jnp.float32),
    compiler_params=pltpu.CompilerParams(use_tc_tiling_on_sc=False),
    scratch_types=[
        pltpu.VMEM((CB,), jnp.int32),
        pltpu.VMEM((CB,), jnp.int32),
        pltpu.VMEM((CB, DEGW), jnp.float32),
        pltpu.VMEM_SHARED((NP, DEGW), jnp.float32),
        pltpu.VMEM_SHARED((NP, DEGW), jnp.float32),
    ],
)
def _degrees(src_hbm, dst_hbm, ones_hbm, zeros_hbm, out_hbm,
             srcc, dstc, ones_v, dego_sp, degi_sp):
    c = lax.axis_index("c")
    s = lax.axis_index("s")
    w = _wid(c, s)
    rs = pl.ds(s * ROWS_PER_SUB, ROWS_PER_SUB)
    pltpu.sync_copy(ones_hbm, ones_v)
    pltpu.sync_copy(zeros_hbm.at[rs], dego_sp.at[rs])
    pltpu.sync_copy(zeros_hbm.at[rs], degi_sp.at[rs])
    plsc.subcore_barrier()

    def step(j, carry):
        pltpu.sync_copy(src_hbm.at[w, j], srcc)
        pltpu.sync_copy(dst_hbm.at[w, j], dstc)
        pltpu.sync_copy(ones_v, dego_sp.at[srcc], add=True)
        pltpu.sync_copy(ones_v, degi_sp.at[dstc], add=True)
        return carry

    lax.fori_loop(0, NCHUNK, step, 0)
    plsc.subcore_barrier()
    pltpu.sync_copy(dego_sp.at[rs], out_hbm.at[c, 0, rs])
    pltpu.sync_copy(degi_sp.at[rs], out_hbm.at[c, 1, rs])


def _make_msgpass(D):
    @functools.partial(
        pl.kernel,
        mesh=_mesh,
        out_type=jax.ShapeDtypeStruct((2, NP, D), jnp.float32),
        compiler_params=pltpu.CompilerParams(use_tc_tiling_on_sc=False),
        scratch_types=[
            pltpu.VMEM((CB,), jnp.int32),
            pltpu.VMEM((CB,), jnp.int32),
            pltpu.VMEM((CB, D), jnp.float32),
            pltpu.VMEM_SHARED((NP, D), jnp.float32),
            pltpu.SemaphoreType.DMA,
        ],
    )
    def _msgpass(h_hbm, src_hbm, dst_hbm, zeros_hbm, out_hbm,
                 srcc, dstc, rows_v, agg_sp, sem):
        c = lax.axis_index("c")
        s = lax.axis_index("s")
        w = _wid(c, s)
        rs = pl.ds(s * ROWS_PER_SUB, ROWS_PER_SUB)
        pltpu.sync_copy(zeros_hbm.at[rs], agg_sp.at[rs])
        plsc.subcore_barrier()

        def step(j, carry):
            pltpu.sync_copy(src_hbm.at[w, j], srcc)
            pltpu.sync_copy(dst_hbm.at[w, j], dstc)
            pltpu.async_copy(h_hbm.at[srcc], rows_v, sem).wait()
            pltpu.sync_copy(rows_v, agg_sp.at[dstc], add=True)
            return carry

        lax.fori_loop(0, NCHUNK, step, 0)
        plsc.subcore_barrier()
        pltpu.sync_copy(agg_sp.at[rs], out_hbm.at[c, rs])

    return _msgpass


_msgpass64 = _make_msgpass(64)
_msgpass32 = _make_msgpass(32)


def _norms_body(deg_ref, norms_ref):
    deg = deg_ref[...]
    dego = deg[0, 0] + deg[1, 0]
    degi = deg[0, 1] + deg[1, 1]
    norms_ref[0] = 1.0 / jnp.sqrt(jnp.maximum(dego, 1.0))
    norms_ref[1] = 1.0 / jnp.sqrt(jnp.maximum(degi, 1.0))


def _norms(degp_packed):
    return pl.pallas_call(
        _norms_body,
        out_shape=jax.ShapeDtypeStruct((2, NP // 128, 128), jnp.float32),
    )(degp_packed)


BR = 640


def _dense_body(x_ref, w_ref, no_ref, o_ref):
    h = jnp.dot(x_ref[...], w_ref[...], preferred_element_type=jnp.float32)
    o_ref[...] = h * no_ref[...]


def _dense_scaled(x, w, norm_out_col):
    K = x.shape[1]
    D = w.shape[1]
    return pl.pallas_call(
        _dense_body,
        grid=(NP // BR,),
        in_specs=[
            pl.BlockSpec((BR, K), lambda i: (i, 0)),
            pl.BlockSpec((K, D), lambda i: (0, 0)),
            pl.BlockSpec((BR, 1), lambda i: (i, 0)),
        ],
        out_specs=pl.BlockSpec((BR, D), lambda i: (i, 0)),
        out_shape=jax.ShapeDtypeStruct((NP, D), jnp.float32),
    )(x, w, norm_out_col)


def _pre_bn_body(aggp_ref, ni_ref, b_ref, o_ref):
    agg = aggp_ref[0] + aggp_ref[1]
    y = agg * ni_ref[...] + b_ref[...]
    o_ref[...] = jnp.maximum(y, 0.0)


def _pre_bn(aggp, norm_in_col, b):
    D = aggp.shape[2]
    return pl.pallas_call(
        _pre_bn_body,
        out_shape=jax.ShapeDtypeStruct((NP, D), jnp.float32),
    )(aggp, norm_in_col, b)


def _bn_apply_body(y_ref, mu_ref, var_ref, g_ref, be_ref, o_ref):
    yv = y_ref[:N]
    hn = (g_ref[...] * (yv - mu_ref[...]) / jnp.sqrt(var_ref[...] + 1e-5)
          + be_ref[...])
    o_ref[...] = jnp.concatenate(
        [hn, jnp.zeros((NP - N, hn.shape[1]), jnp.float32)], axis=0)


def _bn_apply(y, mu, var, g, be):
    D = y.shape[1]
    return pl.pallas_call(
        _bn_apply_body,
        out_shape=jax.ShapeDtypeStruct((NP, D), jnp.float32),
    )(y, mu, var, g, be)


BM = 400


def _decoder_body(ha_ref, hb_ref, o_ref):
    z = lax.dot_general(ha_ref[...], hb_ref[...],
                        (((1,), (1,)), ((), ())),
                        preferred_element_type=jnp.float32)
    o_ref[...] = jax.nn.sigmoid(z)


def _decoder(h):
    return pl.pallas_call(
        _decoder_body,
        grid=(N // BM,),
        in_specs=[
            pl.BlockSpec((BM, 32), lambda i: (i, 0)),
            pl.BlockSpec((N, 32), lambda i: (0, 0)),
        ],
        out_specs=pl.BlockSpec((BM, N), lambda i: (i, 0)),
        out_shape=jax.ShapeDtypeStruct((N, N), jnp.float32),
    )(h, h)


def kernel(features, edge_index, W1, b1, gamma1, beta1, W2, b2, gamma2, beta2,
           Wr, br):
    src = edge_index[0]
    dst = edge_index[1]
    pad = jnp.full((EP - E,), N, dtype=jnp.int32)
    src3 = jnp.concatenate([src, pad]).reshape(NTILES, NCHUNK, CB)
    dst3 = jnp.concatenate([dst, pad]).reshape(NTILES, NCHUNK, CB)
    ones_deg = jnp.ones((CB, DEGW), jnp.float32)
    zeros_deg = jnp.zeros((NP, DEGW), jnp.float32)
    zeros64 = jnp.zeros((NP, 64), jnp.float32)
    zeros32 = jnp.zeros((NP, 32), jnp.float32)
    xpad = jnp.concatenate(
        [features, jnp.zeros((NP - N, features.shape[1]), jnp.float32)], axis=0)

    degp = _degrees(src3, dst3, ones_deg, zeros_deg)
    degp_packed = degp[:, :, :, 0].reshape(2, 2, NP // 128, 128)
    norms_p = _norms(degp_packed)
    no_col = norms_p[0].reshape(NP, 1)
    ni_col = norms_p[1].reshape(NP, 1)

    h1s = _dense_scaled(xpad, W1, no_col)
    aggp1 = _msgpass64(h1s, src3, dst3, zeros64)
    y1 = _pre_bn(aggp1, ni_col, b1.reshape(1, -1))
    mu1 = jnp.mean(y1[:N], axis=0, keepdims=True)
    var1 = jnp.var(y1[:N], axis=0, keepdims=True)
    hn1 = _bn_apply(y1, mu1, var1, gamma1.reshape(1, -1), beta1.reshape(1, -1))
    h2s = _dense_scaled(hn1, W2, no_col)
    aggp2 = _msgpass32(h2s, src3, dst3, zeros32)
    y2 = _pre_bn(aggp2, ni_col, b2.reshape(1, -1))
    mu2 = jnp.mean(y2[:N], axis=0, keepdims=True)
    var2 = jnp.var(y2[:N], axis=0, keepdims=True)
    hn2 = _bn_apply(y2, mu2, var2, gamma2.reshape(1, -1), beta2.reshape(1, -1))
    h = hn2[:N]
    pred = jnp.mean(h, axis=0, keepdims=True) @ Wr + br
    adj = _decoder(h)
    return (adj, pred)
